# Initial kernel scaffold; baseline (speedup 1.0000x reference)
import jax
import jax.numpy as jnp
from jax.experimental import pallas as pl


def kernel(x_bnd, x_res, x_atm_l1, edge_index_bnd, edge_index_atm, edge_index_res, r2a, params):
    raise NotImplementedError("write your pallas kernel here")


# speedup vs baseline: 13.7241x; 13.7241x over previous
import functools

import jax
import jax.numpy as jnp
import numpy as np
from jax import lax
from jax.experimental import pallas as pl
from jax.experimental.pallas import tpu as pltpu
from jax.experimental.pallas import tpu_sc as plsc

N_ATM = 10000
N_RES = 1000
NA_PAD = 10240
NR_PAD = 1024
NC, NS, LANES = 2, 16, 16
NW = NC * NS


def _edge_attn_sc(n_pad, feat, n_chunks, chunk):
    fp1 = feat + 1
    inv_sqrt = 1.0 / np.sqrt(float(feat))
    mesh = plsc.VectorSubcoreMesh(core_axis_name="c", subcore_axis_name="s",
                                  num_cores=NC, num_subcores=NS)
    assert n_chunks % 2 == 0 and chunk % LANES == 0

    @functools.partial(
        pl.kernel,
        mesh=mesh,
        compiler_params=pltpu.CompilerParams(needs_layout_passes=False,
                                             use_tc_tiling_on_sc=False),
        out_type=jax.ShapeDtypeStruct((NC, n_pad, fp1), jnp.float32),
        scratch_types=[
            pltpu.VMEM((n_chunks, chunk), jnp.int32),
            pltpu.VMEM((n_chunks, chunk), jnp.int32),
            pltpu.VMEM((chunk, feat), jnp.float32),
            pltpu.VMEM((chunk, feat), jnp.float32),
            pltpu.VMEM((chunk, feat), jnp.float32),
            pltpu.VMEM((chunk, feat), jnp.float32),
            pltpu.VMEM((chunk, fp1), jnp.float32),
            pltpu.VMEM_SHARED((n_pad, fp1), jnp.float32),
            pltpu.SemaphoreType.DMA,
            pltpu.SemaphoreType.DMA,
            pltpu.SemaphoreType.DMA,
            pltpu.SemaphoreType.DMA,
        ],
    )
    def k(xm_hbm, xq_hbm, src_hbm, dst_hbm, zeros_hbm, out_hbm,
          src_v, dst_v, m0, m1, q0, q1, o_v, acc, sm0, sm1, sq0, sq1):
        cid = lax.axis_index("c")
        sid = lax.axis_index("s")
        wid = sid * NC + cid
        m_bufs, q_bufs = (m0, m1), (q0, q1)
        m_sems, q_sems = (sm0, sm1), (sq0, sq1)

        @pl.when(sid == 0)
        def _():
            pltpu.sync_copy(zeros_hbm, acc)

        pltpu.sync_copy(src_hbm.at[pl.ds(wid * n_chunks, n_chunks)], src_v)
        pltpu.sync_copy(dst_hbm.at[pl.ds(wid * n_chunks, n_chunks)], dst_v)
        plsc.subcore_barrier()

        def issue(g, b):
            @pl.when(g < n_chunks)
            def _():
                pltpu.async_copy(xm_hbm.at[src_v.at[g]], m_bufs[b], m_sems[b])
                pltpu.async_copy(xq_hbm.at[dst_v.at[g]], q_bufs[b], q_sems[b])

        issue(jnp.int32(0), 0)
        issue(jnp.int32(1), 1)

        def body(gp, _):
            for b in range(2):
                g = gp * 2 + b
                pltpu.make_async_copy(
                    xm_hbm.at[src_v.at[g]], m_bufs[b], m_sems[b]).wait()
                pltpu.make_async_copy(
                    xq_hbm.at[dst_v.at[g]], q_bufs[b], q_sems[b]).wait()
                m_v, q_v = m_bufs[b], q_bufs[b]
                for t in range(chunk // LANES):
                    rows = lax.iota(jnp.int32, LANES) + t * LANES
                    accs = [jnp.zeros((LANES,), jnp.float32) for _ in range(4)]
                    for kk in range(feat):
                        col = jnp.full((LANES,), kk, jnp.int32)
                        mk = plsc.load_gather(m_v, [rows, col])
                        qk = plsc.load_gather(q_v, [rows, col])
                        accs[kk % 4] = accs[kk % 4] + mk * qk
                    logit = (accs[0] + accs[1]) + (accs[2] + accs[3])
                    ex = jnp.exp(logit * inv_sqrt)
                    for kk in range(feat):
                        col = jnp.full((LANES,), kk, jnp.int32)
                        mk = plsc.load_gather(m_v, [rows, col])
                        plsc.store_scatter(o_v, [rows, col], ex * mk)
                    colF = jnp.full((LANES,), feat, jnp.int32)
                    plsc.store_scatter(o_v, [rows, colF], ex)
                pltpu.sync_copy(o_v, acc.at[dst_v.at[g]], add=True)
                issue(g + 2, b)
            return 0

        lax.fori_loop(0, n_chunks // 2, body, 0)
        plsc.subcore_barrier()

        @pl.when(sid == 0)
        def _():
            pltpu.sync_copy(acc, out_hbm.at[cid])

    return k


def _elu(v):
    return jnp.where(v > 0, v, jnp.exp(jnp.minimum(v, 0.0)) - 1.0)


def _row_specs(m, bm, shapes):
    return [pl.BlockSpec((bm,) + s[1:], lambda i: (i,) + (0,) * (len(s) - 1))
            if s[0] == m else
            pl.BlockSpec(s, lambda i: (0,) * len(s))
            for s in shapes]


def _mlp2(x, w1, b1, w2, b2, bm=1024):
    m = x.shape[0]

    def body(x_r, w1_r, b1_r, w2_r, b2_r, o_r):
        h = _elu(jnp.dot(x_r[...], w1_r[...],
                         preferred_element_type=jnp.float32) + b1_r[...])
        o_r[...] = jnp.dot(h, w2_r[...],
                           preferred_element_type=jnp.float32) + b2_r[...]

    shapes = [x.shape, w1.shape, (1, b1.shape[0]), w2.shape, (1, b2.shape[0])]
    return pl.pallas_call(
        body,
        grid=(m // bm,),
        in_specs=_row_specs(m, bm, shapes),
        out_specs=pl.BlockSpec((bm, w2.shape[1]), lambda i: (i, 0)),
        out_shape=jax.ShapeDtypeStruct((m, w2.shape[1]), jnp.float32),
    )(x, w1, b1[None, :], w2, b2[None, :])


def _lin_elu(x, w, b, bm=1024):
    m = x.shape[0]

    def body(x_r, w_r, b_r, o_r):
        o_r[...] = _elu(jnp.dot(x_r[...], w_r[...],
                                preferred_element_type=jnp.float32) + b_r[...])

    shapes = [x.shape, w.shape, (1, b.shape[0])]
    return pl.pallas_call(
        body,
        grid=(m // bm,),
        in_specs=_row_specs(m, bm, shapes),
        out_specs=pl.BlockSpec((bm, w.shape[1]), lambda i: (i, 0)),
        out_shape=jax.ShapeDtypeStruct((m, w.shape[1]), jnp.float32),
    )(x, w, b[None, :])


def _premix(x, wm, wq, bm=1024):
    m = x.shape[0]
    fo = wm.shape[1]

    def body(x_r, wm_r, wq_r, om_r, oq_r):
        xv = x_r[...]
        om_r[...] = jnp.dot(xv, wm_r[...], preferred_element_type=jnp.float32)
        oq_r[...] = jnp.dot(xv, wq_r[...], preferred_element_type=jnp.float32)

    shapes = [x.shape, wm.shape, wq.shape]
    out = pl.pallas_call(
        body,
        grid=(m // bm,),
        in_specs=_row_specs(m, bm, shapes),
        out_specs=[pl.BlockSpec((bm, fo), lambda i: (i, 0))] * 2,
        out_shape=[jax.ShapeDtypeStruct((m, fo), jnp.float32)] * 2,
    )(x, wm, wq)
    return out[0], out[1]


def _post(sc_out, x, wo, g, be, bm=1024):
    m, feat = x.shape

    def body(sc_r, x_r, wo_r, g_r, be_r, o_r):
        acc = sc_r[0] + sc_r[1]
        agg = acc[:, :feat] / (acc[:, feat:feat + 1] + 1e-9)
        h = x_r[...] + jnp.dot(agg, wo_r[...],
                               preferred_element_type=jnp.float32)
        mu = jnp.mean(h, axis=-1, keepdims=True)
        var = jnp.mean((h - mu) * (h - mu), axis=-1, keepdims=True)
        o_r[...] = g_r[...] * (h - mu) / jnp.sqrt(var + 1e-5) + be_r[...]

    return pl.pallas_call(
        body,
        grid=(m // bm,),
        in_specs=[
            pl.BlockSpec((2, bm, feat + 1), lambda i: (0, i, 0)),
            pl.BlockSpec((bm, feat), lambda i: (i, 0)),
            pl.BlockSpec(wo.shape, lambda i: (0, 0)),
            pl.BlockSpec((1, feat), lambda i: (0, 0)),
            pl.BlockSpec((1, feat), lambda i: (0, 0)),
        ],
        out_specs=pl.BlockSpec((bm, feat), lambda i: (i, 0)),
        out_shape=jax.ShapeDtypeStruct((m, feat), jnp.float32),
    )(sc_out, x, wo, g[None, :], be[None, :])


def _atm_embed(l0, xl1, w1, b1, w2, b2, wl1, bl1, bm=1024):
    m = l0.shape[0]

    def body(l0_r, xl1_r, w1_r, b1_r, w2_r, b2_r, wl1_r, bl1_r, o_r):
        h = _elu(jnp.dot(l0_r[...], w1_r[...],
                         preferred_element_type=jnp.float32) + b1_r[...])
        h = jnp.dot(h, w2_r[...], preferred_element_type=jnp.float32) + b2_r[...]
        xv = xl1_r[...]
        nrm = jnp.sqrt(jnp.sum(xv * xv, axis=-1, keepdims=True))
        o_r[...] = h + nrm * wl1_r[...] + bl1_r[...]

    shapes = [l0.shape, xl1.shape, w1.shape, (1, b1.shape[0]), w2.shape,
              (1, b2.shape[0]), wl1.shape, (1, bl1.shape[0])]
    return pl.pallas_call(
        body,
        grid=(m // bm,),
        in_specs=_row_specs(m, bm, shapes),
        out_specs=pl.BlockSpec((bm, w2.shape[1]), lambda i: (i, 0)),
        out_shape=jax.ShapeDtypeStruct((m, w2.shape[1]), jnp.float32),
    )(l0, xl1, w1, b1[None, :], w2, b2[None, :], wl1, bl1[None, :])


def _r2a_mm(r2a_pad, h_res, bm=1024):
    m = r2a_pad.shape[0]
    kdim, feat = h_res.shape

    def body(a_r, h_r, o_r):
        o_r[...] = jnp.dot(a_r[...], h_r[...],
                           preferred_element_type=jnp.float32)

    return pl.pallas_call(
        body,
        grid=(m // bm,),
        in_specs=[pl.BlockSpec((bm, kdim), lambda i: (i, 0)),
                  pl.BlockSpec((kdim, feat), lambda i: (0, 0))],
        out_specs=pl.BlockSpec((bm, feat), lambda i: (i, 0)),
        out_shape=jax.ShapeDtypeStruct((m, feat), jnp.float32),
    )(r2a_pad, h_res)


def _a2r_mm(r2a_pad, h_atm, bk=1024):
    ka, nr = r2a_pad.shape
    feat = h_atm.shape[1]

    def body(a_r, h_r, o_r, acc_r):
        i = pl.program_id(0)

        @pl.when(i == 0)
        def _():
            acc_r[...] = jnp.zeros_like(acc_r)

        blk = a_r[...]
        hv = h_r[...]
        ones = jnp.ones((blk.shape[0], 1), jnp.float32)
        rhs = jnp.concatenate([hv, ones], axis=1)
        acc_r[...] += jax.lax.dot_general(
            blk, rhs, (((0,), (0,)), ((), ())),
            preferred_element_type=jnp.float32)

        @pl.when(i == (ka // bk) - 1)
        def _():
            acc = acc_r[...]
            o_r[...] = acc[:, :feat] / (acc[:, feat:feat + 1] + 1.0)

    return pl.pallas_call(
        body,
        grid=(ka // bk,),
        in_specs=[pl.BlockSpec((bk, nr), lambda i: (i, 0)),
                  pl.BlockSpec((bk, feat), lambda i: (i, 0))],
        out_specs=pl.BlockSpec((nr, feat), lambda i: (0, 0)),
        out_shape=jax.ShapeDtypeStruct((nr, feat), jnp.float32),
        scratch_shapes=[pltpu.VMEM((nr, feat + 1), jnp.float32)],
    )(r2a_pad, h_atm)


def _head(sc_out, h_atm, wskip, wc1, bc1, wc2, bc2, ww1, bw1, ww2, bw2):
    m = h_atm.shape[0]

    def body(sc_r, ha_r, wsk_r, wc1_r, bc1_r, wc2_r, bc2_r,
             ww1_r, bw1_r, ww2_r, bw2_r, o_r):
        acc = sc_r[0] + sc_r[1]
        h64 = acc[:, :64] / (acc[:, 64:65] + 1e-9)
        h64 = h64 + jnp.dot(ha_r[...], wsk_r[...],
                            preferred_element_type=jnp.float32)
        hc = jnp.maximum(jnp.dot(h64, wc1_r[...],
                                 preferred_element_type=jnp.float32)
                         + bc1_r[...], 0.0)
        c = jnp.dot(hc, wc2_r[...], preferred_element_type=jnp.float32) + bc2_r[...]
        hw = jnp.maximum(jnp.dot(h64, ww1_r[...],
                                 preferred_element_type=jnp.float32)
                         + bw1_r[...], 0.0)
        wl = jnp.dot(hw, ww2_r[...], preferred_element_type=jnp.float32) + bw2_r[...]
        rows = lax.broadcasted_iota(jnp.int32, (m, 1), 0)
        wl = jnp.where(rows < N_ATM, wl, -1e30)
        mx = jnp.max(wl)
        ex = jnp.exp(wl - mx)
        den = jnp.sum(ex)
        o_r[...] = jnp.sum(ex * c, axis=0, keepdims=True) / den

    return pl.pallas_call(
        body,
        in_specs=[pl.BlockSpec(sc_out.shape, lambda: (0, 0, 0))]
        + [pl.BlockSpec(s, lambda: (0,) * len(s))
           for s in [h_atm.shape, wskip.shape, wc1.shape, (1, 64), wc2.shape,
                     (1, 2), ww1.shape, (1, 64), ww2.shape, (1, 1)]],
        out_specs=pl.BlockSpec((1, 2), lambda: (0, 0)),
        out_shape=jax.ShapeDtypeStruct((1, 2), jnp.float32),
    )(sc_out, h_atm, wskip, wc1, bc1[None, :], wc2, bc2[None, :],
      ww1, bw1[None, :], ww2, bw2[None, :])


def _pad_rows(x, n):
    return jnp.pad(x, ((0, n - x.shape[0]),) + ((0, 0),) * (x.ndim - 1))


def _pad_edges(ei, dummy, n_chunks, chunk):
    e_pad = NW * n_chunks * chunk
    src = jnp.pad(ei[0], (0, e_pad - ei.shape[1]), constant_values=dummy)
    dst = jnp.pad(ei[1], (0, e_pad - ei.shape[1]), constant_values=dummy)
    return src.reshape(NW * n_chunks, chunk), dst.reshape(NW * n_chunks, chunk)


def _n_chunks(e, chunk):
    n = -(-e // (NW * chunk))
    return -(-n // 8) * 8


_SC_ATM32 = _edge_attn_sc(NA_PAD, 32, _n_chunks(640000, 128), 128)
_SC_RES32 = _edge_attn_sc(NR_PAD, 32, _n_chunks(16000, 128), 128)
_SC_ATM64 = _edge_attn_sc(NA_PAD, 64, _n_chunks(640000, 64), 64)


def _mp_layer(x, src2d, dst2d, lp, zeros, sc_fn):
    xm, xq = _premix(x, lp['Wm'], lp['Wq'])
    sc_out = sc_fn(xm, xq, src2d, dst2d, zeros)
    return _post(sc_out, x, lp['Wo'], lp['g'], lp['be'])


def kernel(x_bnd, x_res, x_atm_l1, edge_index_bnd, edge_index_atm,
           edge_index_res, r2a, params):
    p = params
    xb = _pad_rows(x_bnd, NA_PAD)
    xr = _pad_rows(x_res, NR_PAD)
    xl1 = _pad_rows(x_atm_l1.reshape(N_ATM, 3), NA_PAD)
    r2ap = jnp.pad(r2a, ((0, NA_PAD - N_ATM), (0, NR_PAD - N_RES)))

    srcb, dstb = _pad_edges(edge_index_bnd, N_ATM, _n_chunks(640000, 128), 128)
    srca, dsta = _pad_edges(edge_index_atm, N_ATM, _n_chunks(640000, 128), 128)
    srcr, dstr = _pad_edges(edge_index_res, N_RES, _n_chunks(16000, 128), 128)
    srcf, dstf = _pad_edges(edge_index_atm, N_ATM, _n_chunks(640000, 64), 64)

    z33a = jnp.zeros((NA_PAD, 33), jnp.float32)
    z33r = jnp.zeros((NR_PAD, 33), jnp.float32)
    z65a = jnp.zeros((NA_PAD, 65), jnp.float32)

    h_bnd = _mlp2(xb, p['W1_bnd'], p['b1_bnd'], p['W2_bnd'], p['b2_bnd'])
    for lp in p['bnd_layers']:
        h_bnd = _mp_layer(h_bnd, srcb, dstb, lp, z33a, _SC_ATM32)

    h_res = _mlp2(xr, p['W1_res'], p['b1_res'], p['W2_res'], p['b2_res'])
    h_resA = _r2a_mm(r2ap, h_res)
    l0 = jnp.concatenate([h_bnd, h_resA], axis=1)
    h_atm = _atm_embed(l0, xl1, p['W1_atm'], p['b1_atm'], p['W2_atm'],
                       p['b2_atm'], p['Wl1'], p['bl1'])

    for i, (lpa, lpr) in enumerate(zip(p['atm_layers'], p['res_layers'])):
        h_atm = _mp_layer(h_atm, srca, dsta, lpa, z33a, _SC_ATM32)
        h_res = _mp_layer(h_res, srcr, dstr, lpr, z33r, _SC_RES32)
        if i % 2 == 1:
            hA = jnp.concatenate([h_atm, _r2a_mm(r2ap, h_res)], axis=1)
            hR = jnp.concatenate([h_res, _a2r_mm(r2ap, h_atm)], axis=1)
            h_atm = _lin_elu(hA, p['Wla'], p['bla'])
            h_res = _lin_elu(hR, p['Wlr'], p['blr'])

    xm64, xq64 = _premix(h_atm, p['fin']['Wm'], p['fin']['Wq'])
    sc65 = _SC_ATM64(xm64, xq64, srcf, dstf, z65a)
    out = _head(sc65, h_atm, p['Wskip'], p['Wc1'], p['bc1'], p['Wc2'],
                p['bc2'], p['Ww1'], p['bw1'], p['Ww2'], p['bw2'])
    return out.reshape(2)


# speedup vs baseline: 20.1762x; 1.4701x over previous
import functools

import jax
import jax.numpy as jnp
import numpy as np
from jax import lax
from jax.experimental import pallas as pl
from jax.experimental.pallas import tpu as pltpu
from jax.experimental.pallas import tpu_sc as plsc

N_ATM = 10000
N_RES = 1000
NA_PAD = 10240
NR_PAD = 1024
NC, NS, LANES = 2, 16, 16
NW = NC * NS


def _edge_attn_sc(n_pad, feat, n_chunks, chunk):
    fp1 = feat + 1
    inv_sqrt = 1.0 / np.sqrt(float(feat))
    mesh = plsc.VectorSubcoreMesh(core_axis_name="c", subcore_axis_name="s",
                                  num_cores=NC, num_subcores=NS)
    nbuf = 4
    cache_cols = feat <= 32
    assert n_chunks % nbuf == 0 and chunk % LANES == 0

    @functools.partial(
        pl.kernel,
        mesh=mesh,
        compiler_params=pltpu.CompilerParams(needs_layout_passes=False,
                                             use_tc_tiling_on_sc=False),
        out_type=jax.ShapeDtypeStruct((NC, n_pad, fp1), jnp.float32),
        scratch_types=(
            [pltpu.VMEM((n_chunks, chunk), jnp.int32)] * 2
            + [pltpu.VMEM((chunk, feat), jnp.float32)] * (2 * nbuf)
            + [pltpu.VMEM((chunk, fp1), jnp.float32)] * 2
            + [pltpu.VMEM_SHARED((n_pad, fp1), jnp.float32)]
            + [pltpu.SemaphoreType.DMA] * (2 * nbuf + 2)
        ),
    )
    def k(xm_hbm, xq_hbm, src_hbm, dst_hbm, zeros_hbm, out_hbm, *refs):
        src_v, dst_v = refs[0], refs[1]
        m_bufs = refs[2:2 + nbuf]
        q_bufs = refs[2 + nbuf:2 + 2 * nbuf]
        o_bufs = refs[2 + 2 * nbuf:4 + 2 * nbuf]
        acc = refs[4 + 2 * nbuf]
        m_sems = refs[5 + 2 * nbuf:5 + 3 * nbuf]
        q_sems = refs[5 + 3 * nbuf:5 + 4 * nbuf]
        s_sems = refs[5 + 4 * nbuf:7 + 4 * nbuf]
        cid = lax.axis_index("c")
        sid = lax.axis_index("s")
        wid = sid * NC + cid

        @pl.when(sid == 0)
        def _():
            pltpu.sync_copy(zeros_hbm, acc)

        pltpu.sync_copy(src_hbm.at[pl.ds(wid * n_chunks, n_chunks)], src_v)
        pltpu.sync_copy(dst_hbm.at[pl.ds(wid * n_chunks, n_chunks)], dst_v)
        plsc.subcore_barrier()

        def issue(g, b):
            @pl.when(g < n_chunks)
            def _():
                pltpu.async_copy(xm_hbm.at[src_v.at[g]], m_bufs[b], m_sems[b])
                pltpu.async_copy(xq_hbm.at[dst_v.at[g]], q_bufs[b], q_sems[b])

        for b in range(nbuf):
            issue(jnp.int32(b), b)

        def body(gp, _):
            for j in range(nbuf):
                g = gp * nbuf + j
                ob = j % 2
                o_v = o_bufs[ob]
                pltpu.make_async_copy(
                    xm_hbm.at[src_v.at[g]], m_bufs[j], m_sems[j]).wait()
                pltpu.make_async_copy(
                    xq_hbm.at[dst_v.at[g]], q_bufs[j], q_sems[j]).wait()

                @pl.when(g >= 2)
                def _():
                    pltpu.make_async_copy(
                        o_v, acc.at[dst_v.at[g]], s_sems[ob]).wait()

                m_v, q_v = m_bufs[j], q_bufs[j]

                def tbody(t, _, m_v=m_v, q_v=q_v, o_v=o_v):
                    rows = lax.iota(jnp.int32, LANES) + t * LANES
                    accs = [jnp.zeros((LANES,), jnp.float32) for _ in range(4)]
                    mcols = []
                    for kk in range(feat):
                        col = jnp.full((LANES,), kk, jnp.int32)
                        mk = plsc.load_gather(m_v, [rows, col])
                        qk = plsc.load_gather(q_v, [rows, col])
                        accs[kk % 4] = accs[kk % 4] + mk * qk
                        if cache_cols:
                            mcols.append(mk)
                    logit = (accs[0] + accs[1]) + (accs[2] + accs[3])
                    ex = jnp.exp(logit * inv_sqrt)
                    for kk in range(feat):
                        col = jnp.full((LANES,), kk, jnp.int32)
                        mk = (mcols[kk] if cache_cols
                              else plsc.load_gather(m_v, [rows, col]))
                        plsc.store_scatter(o_v, [rows, col], ex * mk)
                    colF = jnp.full((LANES,), feat, jnp.int32)
                    plsc.store_scatter(o_v, [rows, colF], ex)
                    return 0

                lax.fori_loop(0, chunk // LANES, tbody, 0)
                issue(g + nbuf, j)
                pltpu.async_copy(o_v, acc.at[dst_v.at[g]], s_sems[ob],
                                 add=True)
            return 0

        lax.fori_loop(0, n_chunks // nbuf, body, 0)
        for ob in range(2):
            pltpu.make_async_copy(
                o_bufs[ob], acc.at[dst_v.at[0]], s_sems[ob]).wait()
        plsc.subcore_barrier()

        @pl.when(sid == 0)
        def _():
            pltpu.sync_copy(acc, out_hbm.at[cid])

    return k


def _elu(v):
    return jnp.where(v > 0, v, jnp.exp(jnp.minimum(v, 0.0)) - 1.0)


def _row_specs(m, bm, shapes):
    return [pl.BlockSpec((bm,) + s[1:], lambda i: (i,) + (0,) * (len(s) - 1))
            if s[0] == m else
            pl.BlockSpec(s, lambda i: (0,) * len(s))
            for s in shapes]


def _mlp2(x, w1, b1, w2, b2, bm=1024):
    m = x.shape[0]

    def body(x_r, w1_r, b1_r, w2_r, b2_r, o_r):
        h = _elu(jnp.dot(x_r[...], w1_r[...],
                         preferred_element_type=jnp.float32) + b1_r[...])
        o_r[...] = jnp.dot(h, w2_r[...],
                           preferred_element_type=jnp.float32) + b2_r[...]

    shapes = [x.shape, w1.shape, (1, b1.shape[0]), w2.shape, (1, b2.shape[0])]
    return pl.pallas_call(
        body,
        grid=(m // bm,),
        in_specs=_row_specs(m, bm, shapes),
        out_specs=pl.BlockSpec((bm, w2.shape[1]), lambda i: (i, 0)),
        out_shape=jax.ShapeDtypeStruct((m, w2.shape[1]), jnp.float32),
    )(x, w1, b1[None, :], w2, b2[None, :])


def _lin_elu(x, w, b, bm=1024):
    m = x.shape[0]

    def body(x_r, w_r, b_r, o_r):
        o_r[...] = _elu(jnp.dot(x_r[...], w_r[...],
                                preferred_element_type=jnp.float32) + b_r[...])

    shapes = [x.shape, w.shape, (1, b.shape[0])]
    return pl.pallas_call(
        body,
        grid=(m // bm,),
        in_specs=_row_specs(m, bm, shapes),
        out_specs=pl.BlockSpec((bm, w.shape[1]), lambda i: (i, 0)),
        out_shape=jax.ShapeDtypeStruct((m, w.shape[1]), jnp.float32),
    )(x, w, b[None, :])


def _premix(x, wm, wq, bm=1024):
    m = x.shape[0]
    fo = wm.shape[1]

    def body(x_r, wm_r, wq_r, om_r, oq_r):
        xv = x_r[...]
        om_r[...] = jnp.dot(xv, wm_r[...], preferred_element_type=jnp.float32)
        oq_r[...] = jnp.dot(xv, wq_r[...], preferred_element_type=jnp.float32)

    shapes = [x.shape, wm.shape, wq.shape]
    out = pl.pallas_call(
        body,
        grid=(m // bm,),
        in_specs=_row_specs(m, bm, shapes),
        out_specs=[pl.BlockSpec((bm, fo), lambda i: (i, 0))] * 2,
        out_shape=[jax.ShapeDtypeStruct((m, fo), jnp.float32)] * 2,
    )(x, wm, wq)
    return out[0], out[1]


def _post(sc_out, x, wo, g, be, bm=1024):
    m, feat = x.shape

    def body(sc_r, x_r, wo_r, g_r, be_r, o_r):
        acc = sc_r[0] + sc_r[1]
        agg = acc[:, :feat] / (acc[:, feat:feat + 1] + 1e-9)
        h = x_r[...] + jnp.dot(agg, wo_r[...],
                               preferred_element_type=jnp.float32)
        mu = jnp.mean(h, axis=-1, keepdims=True)
        var = jnp.mean((h - mu) * (h - mu), axis=-1, keepdims=True)
        o_r[...] = g_r[...] * (h - mu) / jnp.sqrt(var + 1e-5) + be_r[...]

    return pl.pallas_call(
        body,
        grid=(m // bm,),
        in_specs=[
            pl.BlockSpec((2, bm, feat + 1), lambda i: (0, i, 0)),
            pl.BlockSpec((bm, feat), lambda i: (i, 0)),
            pl.BlockSpec(wo.shape, lambda i: (0, 0)),
            pl.BlockSpec((1, feat), lambda i: (0, 0)),
            pl.BlockSpec((1, feat), lambda i: (0, 0)),
        ],
        out_specs=pl.BlockSpec((bm, feat), lambda i: (i, 0)),
        out_shape=jax.ShapeDtypeStruct((m, feat), jnp.float32),
    )(sc_out, x, wo, g[None, :], be[None, :])


def _atm_embed(l0, xl1, w1, b1, w2, b2, wl1, bl1, bm=1024):
    m = l0.shape[0]

    def body(l0_r, xl1_r, w1_r, b1_r, w2_r, b2_r, wl1_r, bl1_r, o_r):
        h = _elu(jnp.dot(l0_r[...], w1_r[...],
                         preferred_element_type=jnp.float32) + b1_r[...])
        h = jnp.dot(h, w2_r[...], preferred_element_type=jnp.float32) + b2_r[...]
        xv = xl1_r[...]
        nrm = jnp.sqrt(jnp.sum(xv * xv, axis=-1, keepdims=True))
        o_r[...] = h + nrm * wl1_r[...] + bl1_r[...]

    shapes = [l0.shape, xl1.shape, w1.shape, (1, b1.shape[0]), w2.shape,
              (1, b2.shape[0]), wl1.shape, (1, bl1.shape[0])]
    return pl.pallas_call(
        body,
        grid=(m // bm,),
        in_specs=_row_specs(m, bm, shapes),
        out_specs=pl.BlockSpec((bm, w2.shape[1]), lambda i: (i, 0)),
        out_shape=jax.ShapeDtypeStruct((m, w2.shape[1]), jnp.float32),
    )(l0, xl1, w1, b1[None, :], w2, b2[None, :], wl1, bl1[None, :])


def _r2a_mm(r2a_pad, h_res, bm=1024):
    m = r2a_pad.shape[0]
    kdim, feat = h_res.shape

    def body(a_r, h_r, o_r):
        o_r[...] = jnp.dot(a_r[...], h_r[...],
                           preferred_element_type=jnp.float32)

    return pl.pallas_call(
        body,
        grid=(m // bm,),
        in_specs=[pl.BlockSpec((bm, kdim), lambda i: (i, 0)),
                  pl.BlockSpec((kdim, feat), lambda i: (0, 0))],
        out_specs=pl.BlockSpec((bm, feat), lambda i: (i, 0)),
        out_shape=jax.ShapeDtypeStruct((m, feat), jnp.float32),
    )(r2a_pad, h_res)


def _a2r_mm(r2a_pad, h_atm, bk=1024):
    ka, nr = r2a_pad.shape
    feat = h_atm.shape[1]

    def body(a_r, h_r, o_r, acc_r):
        i = pl.program_id(0)

        @pl.when(i == 0)
        def _():
            acc_r[...] = jnp.zeros_like(acc_r)

        blk = a_r[...]
        hv = h_r[...]
        ones = jnp.ones((blk.shape[0], 1), jnp.float32)
        rhs = jnp.concatenate([hv, ones], axis=1)
        acc_r[...] += jax.lax.dot_general(
            blk, rhs, (((0,), (0,)), ((), ())),
            preferred_element_type=jnp.float32)

        @pl.when(i == (ka // bk) - 1)
        def _():
            acc = acc_r[...]
            o_r[...] = acc[:, :feat] / (acc[:, feat:feat + 1] + 1.0)

    return pl.pallas_call(
        body,
        grid=(ka // bk,),
        in_specs=[pl.BlockSpec((bk, nr), lambda i: (i, 0)),
                  pl.BlockSpec((bk, feat), lambda i: (i, 0))],
        out_specs=pl.BlockSpec((nr, feat), lambda i: (0, 0)),
        out_shape=jax.ShapeDtypeStruct((nr, feat), jnp.float32),
        scratch_shapes=[pltpu.VMEM((nr, feat + 1), jnp.float32)],
    )(r2a_pad, h_atm)


def _head(sc_out, h_atm, wskip, wc1, bc1, wc2, bc2, ww1, bw1, ww2, bw2):
    m = h_atm.shape[0]

    def body(sc_r, ha_r, wsk_r, wc1_r, bc1_r, wc2_r, bc2_r,
             ww1_r, bw1_r, ww2_r, bw2_r, o_r):
        acc = sc_r[0] + sc_r[1]
        h64 = acc[:, :64] / (acc[:, 64:65] + 1e-9)
        h64 = h64 + jnp.dot(ha_r[...], wsk_r[...],
                            preferred_element_type=jnp.float32)
        hc = jnp.maximum(jnp.dot(h64, wc1_r[...],
                                 preferred_element_type=jnp.float32)
                         + bc1_r[...], 0.0)
        c = jnp.dot(hc, wc2_r[...], preferred_element_type=jnp.float32) + bc2_r[...]
        hw = jnp.maximum(jnp.dot(h64, ww1_r[...],
                                 preferred_element_type=jnp.float32)
                         + bw1_r[...], 0.0)
        wl = jnp.dot(hw, ww2_r[...], preferred_element_type=jnp.float32) + bw2_r[...]
        rows = lax.broadcasted_iota(jnp.int32, (m, 1), 0)
        wl = jnp.where(rows < N_ATM, wl, -1e30)
        mx = jnp.max(wl)
        ex = jnp.exp(wl - mx)
        den = jnp.sum(ex)
        o_r[...] = jnp.sum(ex * c, axis=0, keepdims=True) / den

    return pl.pallas_call(
        body,
        in_specs=[pl.BlockSpec(sc_out.shape, lambda: (0, 0, 0))]
        + [pl.BlockSpec(s, lambda: (0,) * len(s))
           for s in [h_atm.shape, wskip.shape, wc1.shape, (1, 64), wc2.shape,
                     (1, 2), ww1.shape, (1, 64), ww2.shape, (1, 1)]],
        out_specs=pl.BlockSpec((1, 2), lambda: (0, 0)),
        out_shape=jax.ShapeDtypeStruct((1, 2), jnp.float32),
    )(sc_out, h_atm, wskip, wc1, bc1[None, :], wc2, bc2[None, :],
      ww1, bw1[None, :], ww2, bw2[None, :])


def _pad_rows(x, n):
    return jnp.pad(x, ((0, n - x.shape[0]),) + ((0, 0),) * (x.ndim - 1))


def _pad_edges(ei, dummy, n_chunks, chunk):
    e_pad = NW * n_chunks * chunk
    src = jnp.pad(ei[0], (0, e_pad - ei.shape[1]), constant_values=dummy)
    dst = jnp.pad(ei[1], (0, e_pad - ei.shape[1]), constant_values=dummy)
    return src.reshape(NW * n_chunks, chunk), dst.reshape(NW * n_chunks, chunk)


def _n_chunks(e, chunk):
    n = -(-e // (NW * chunk))
    return -(-n // 8) * 8


_SC_ATM32 = _edge_attn_sc(NA_PAD, 32, _n_chunks(640000, 128), 128)
_SC_RES32 = _edge_attn_sc(NR_PAD, 32, _n_chunks(16000, 128), 128)
_SC_ATM64 = _edge_attn_sc(NA_PAD, 64, _n_chunks(640000, 64), 64)


def _mp_layer(x, src2d, dst2d, lp, zeros, sc_fn):
    xm, xq = _premix(x, lp['Wm'], lp['Wq'])
    sc_out = sc_fn(xm, xq, src2d, dst2d, zeros)
    return _post(sc_out, x, lp['Wo'], lp['g'], lp['be'])


def kernel(x_bnd, x_res, x_atm_l1, edge_index_bnd, edge_index_atm,
           edge_index_res, r2a, params):
    p = params
    xb = _pad_rows(x_bnd, NA_PAD)
    xr = _pad_rows(x_res, NR_PAD)
    xl1 = _pad_rows(x_atm_l1.reshape(N_ATM, 3), NA_PAD)
    r2ap = jnp.pad(r2a, ((0, NA_PAD - N_ATM), (0, NR_PAD - N_RES)))

    srcb, dstb = _pad_edges(edge_index_bnd, N_ATM, _n_chunks(640000, 128), 128)
    srca, dsta = _pad_edges(edge_index_atm, N_ATM, _n_chunks(640000, 128), 128)
    srcr, dstr = _pad_edges(edge_index_res, N_RES, _n_chunks(16000, 128), 128)
    srcf, dstf = _pad_edges(edge_index_atm, N_ATM, _n_chunks(640000, 64), 64)

    z33a = jnp.zeros((NA_PAD, 33), jnp.float32)
    z33r = jnp.zeros((NR_PAD, 33), jnp.float32)
    z65a = jnp.zeros((NA_PAD, 65), jnp.float32)

    h_bnd = _mlp2(xb, p['W1_bnd'], p['b1_bnd'], p['W2_bnd'], p['b2_bnd'])
    for lp in p['bnd_layers']:
        h_bnd = _mp_layer(h_bnd, srcb, dstb, lp, z33a, _SC_ATM32)

    h_res = _mlp2(xr, p['W1_res'], p['b1_res'], p['W2_res'], p['b2_res'])
    h_resA = _r2a_mm(r2ap, h_res)
    l0 = jnp.concatenate([h_bnd, h_resA], axis=1)
    h_atm = _atm_embed(l0, xl1, p['W1_atm'], p['b1_atm'], p['W2_atm'],
                       p['b2_atm'], p['Wl1'], p['bl1'])

    for i, (lpa, lpr) in enumerate(zip(p['atm_layers'], p['res_layers'])):
        h_atm = _mp_layer(h_atm, srca, dsta, lpa, z33a, _SC_ATM32)
        h_res = _mp_layer(h_res, srcr, dstr, lpr, z33r, _SC_RES32)
        if i % 2 == 1:
            hA = jnp.concatenate([h_atm, _r2a_mm(r2ap, h_res)], axis=1)
            hR = jnp.concatenate([h_res, _a2r_mm(r2ap, h_atm)], axis=1)
            h_atm = _lin_elu(hA, p['Wla'], p['bla'])
            h_res = _lin_elu(hR, p['Wlr'], p['blr'])

    xm64, xq64 = _premix(h_atm, p['fin']['Wm'], p['fin']['Wq'])
    sc65 = _SC_ATM64(xm64, xq64, srcf, dstf, z65a)
    out = _head(sc65, h_atm, p['Wskip'], p['Wc1'], p['bc1'], p['Wc2'],
                p['bc2'], p['Ww1'], p['bw1'], p['Ww2'], p['bw2'])
    return out.reshape(2)


# speedup vs baseline: 21.3019x; 1.0558x over previous
import functools

import jax
import jax.numpy as jnp
import numpy as np
from jax import lax
from jax.experimental import pallas as pl
from jax.experimental.pallas import tpu as pltpu
from jax.experimental.pallas import tpu_sc as plsc

N_ATM = 10000
N_RES = 1000
NA_PAD = 10240
NR_PAD = 1024
NC, NS, LANES = 2, 16, 16
NW = NC * NS


def _edge_attn_sc(n_pad, feat, n_chunks, chunk):
    fp1 = feat + 1
    inv_sqrt = 1.0 / np.sqrt(float(feat))
    mesh = plsc.VectorSubcoreMesh(core_axis_name="c", subcore_axis_name="s",
                                  num_cores=NC, num_subcores=NS)
    nbuf = 4
    cache_n = 32
    assert n_chunks % nbuf == 0 and chunk % LANES == 0

    @functools.partial(
        pl.kernel,
        mesh=mesh,
        compiler_params=pltpu.CompilerParams(needs_layout_passes=False,
                                             use_tc_tiling_on_sc=False),
        out_type=jax.ShapeDtypeStruct((NC, n_pad, fp1), jnp.float32),
        scratch_types=(
            [pltpu.VMEM((n_chunks, chunk), jnp.int32)] * 2
            + [pltpu.VMEM((chunk, feat), jnp.float32)] * (2 * nbuf)
            + [pltpu.VMEM((chunk, fp1), jnp.float32)] * 2
            + [pltpu.VMEM_SHARED((n_pad, fp1), jnp.float32)]
            + [pltpu.SemaphoreType.DMA] * (2 * nbuf + 2)
        ),
    )
    def k(xm_hbm, xq_hbm, src_hbm, dst_hbm, zeros_hbm, out_hbm, *refs):
        src_v, dst_v = refs[0], refs[1]
        m_bufs = refs[2:2 + nbuf]
        q_bufs = refs[2 + nbuf:2 + 2 * nbuf]
        o_bufs = refs[2 + 2 * nbuf:4 + 2 * nbuf]
        acc = refs[4 + 2 * nbuf]
        m_sems = refs[5 + 2 * nbuf:5 + 3 * nbuf]
        q_sems = refs[5 + 3 * nbuf:5 + 4 * nbuf]
        s_sems = refs[5 + 4 * nbuf:7 + 4 * nbuf]
        cid = lax.axis_index("c")
        sid = lax.axis_index("s")
        wid = sid * NC + cid

        @pl.when(sid == 0)
        def _():
            pltpu.sync_copy(zeros_hbm, acc)

        pltpu.sync_copy(src_hbm.at[pl.ds(wid * n_chunks, n_chunks)], src_v)
        pltpu.sync_copy(dst_hbm.at[pl.ds(wid * n_chunks, n_chunks)], dst_v)
        plsc.subcore_barrier()

        def issue(g, b):
            @pl.when(g < n_chunks)
            def _():
                pltpu.async_copy(xm_hbm.at[src_v.at[g]], m_bufs[b], m_sems[b])
                pltpu.async_copy(xq_hbm.at[dst_v.at[g]], q_bufs[b], q_sems[b])

        for b in range(nbuf):
            issue(jnp.int32(b), b)

        def body(gp, _):
            for j in range(nbuf):
                g = gp * nbuf + j
                ob = j % 2
                o_v = o_bufs[ob]
                pltpu.make_async_copy(
                    xm_hbm.at[src_v.at[g]], m_bufs[j], m_sems[j]).wait()
                pltpu.make_async_copy(
                    xq_hbm.at[dst_v.at[g]], q_bufs[j], q_sems[j]).wait()

                @pl.when(g >= 2)
                def _():
                    pltpu.make_async_copy(
                        o_v, acc.at[dst_v.at[g]], s_sems[ob]).wait()

                m_v, q_v = m_bufs[j], q_bufs[j]

                def tbody(t, _, m_v=m_v, q_v=q_v, o_v=o_v):
                    rows = lax.iota(jnp.int32, LANES) + t * LANES
                    accs = [jnp.zeros((LANES,), jnp.float32) for _ in range(4)]
                    mcols = []
                    for kk in range(feat):
                        col = jnp.full((LANES,), kk, jnp.int32)
                        mk = plsc.load_gather(m_v, [rows, col])
                        qk = plsc.load_gather(q_v, [rows, col])
                        accs[kk % 4] = accs[kk % 4] + mk * qk
                        if kk < cache_n:
                            mcols.append(mk)
                    logit = (accs[0] + accs[1]) + (accs[2] + accs[3])
                    ex = jnp.exp(logit * inv_sqrt)
                    for kk in range(feat):
                        col = jnp.full((LANES,), kk, jnp.int32)
                        mk = (mcols[kk] if kk < cache_n
                              else plsc.load_gather(m_v, [rows, col]))
                        plsc.store_scatter(o_v, [rows, col], ex * mk)
                    colF = jnp.full((LANES,), feat, jnp.int32)
                    plsc.store_scatter(o_v, [rows, colF], ex)
                    return 0

                lax.fori_loop(0, chunk // LANES, tbody, 0)
                issue(g + nbuf, j)
                pltpu.async_copy(o_v, acc.at[dst_v.at[g]], s_sems[ob],
                                 add=True)
            return 0

        lax.fori_loop(0, n_chunks // nbuf, body, 0)
        for ob in range(2):
            pltpu.make_async_copy(
                o_bufs[ob], acc.at[dst_v.at[0]], s_sems[ob]).wait()
        plsc.subcore_barrier()

        @pl.when(sid == 0)
        def _():
            pltpu.sync_copy(acc, out_hbm.at[cid])

    return k


def _elu(v):
    return jnp.where(v > 0, v, jnp.exp(jnp.minimum(v, 0.0)) - 1.0)


def _row_specs(m, bm, shapes):
    return [pl.BlockSpec((bm,) + s[1:], lambda i: (i,) + (0,) * (len(s) - 1))
            if s[0] == m else
            pl.BlockSpec(s, lambda i: (0,) * len(s))
            for s in shapes]


def _mlp2(x, w1, b1, w2, b2, bm=1024):
    m = x.shape[0]

    def body(x_r, w1_r, b1_r, w2_r, b2_r, o_r):
        h = _elu(jnp.dot(x_r[...], w1_r[...],
                         preferred_element_type=jnp.float32) + b1_r[...])
        o_r[...] = jnp.dot(h, w2_r[...],
                           preferred_element_type=jnp.float32) + b2_r[...]

    shapes = [x.shape, w1.shape, (1, b1.shape[0]), w2.shape, (1, b2.shape[0])]
    return pl.pallas_call(
        body,
        grid=(m // bm,),
        in_specs=_row_specs(m, bm, shapes),
        out_specs=pl.BlockSpec((bm, w2.shape[1]), lambda i: (i, 0)),
        out_shape=jax.ShapeDtypeStruct((m, w2.shape[1]), jnp.float32),
    )(x, w1, b1[None, :], w2, b2[None, :])


def _lin_elu(x, w, b, bm=1024):
    m = x.shape[0]

    def body(x_r, w_r, b_r, o_r):
        o_r[...] = _elu(jnp.dot(x_r[...], w_r[...],
                                preferred_element_type=jnp.float32) + b_r[...])

    shapes = [x.shape, w.shape, (1, b.shape[0])]
    return pl.pallas_call(
        body,
        grid=(m // bm,),
        in_specs=_row_specs(m, bm, shapes),
        out_specs=pl.BlockSpec((bm, w.shape[1]), lambda i: (i, 0)),
        out_shape=jax.ShapeDtypeStruct((m, w.shape[1]), jnp.float32),
    )(x, w, b[None, :])


def _premix(x, wm, wq, bm=1024):
    m = x.shape[0]
    fo = wm.shape[1]

    def body(x_r, wm_r, wq_r, om_r, oq_r):
        xv = x_r[...]
        om_r[...] = jnp.dot(xv, wm_r[...], preferred_element_type=jnp.float32)
        oq_r[...] = jnp.dot(xv, wq_r[...], preferred_element_type=jnp.float32)

    shapes = [x.shape, wm.shape, wq.shape]
    out = pl.pallas_call(
        body,
        grid=(m // bm,),
        in_specs=_row_specs(m, bm, shapes),
        out_specs=[pl.BlockSpec((bm, fo), lambda i: (i, 0))] * 2,
        out_shape=[jax.ShapeDtypeStruct((m, fo), jnp.float32)] * 2,
    )(x, wm, wq)
    return out[0], out[1]


def _post(sc_out, x, wo, g, be, bm=1024):
    m, feat = x.shape

    def body(sc_r, x_r, wo_r, g_r, be_r, o_r):
        acc = sc_r[0] + sc_r[1]
        agg = acc[:, :feat] / (acc[:, feat:feat + 1] + 1e-9)
        h = x_r[...] + jnp.dot(agg, wo_r[...],
                               preferred_element_type=jnp.float32)
        mu = jnp.mean(h, axis=-1, keepdims=True)
        var = jnp.mean((h - mu) * (h - mu), axis=-1, keepdims=True)
        o_r[...] = g_r[...] * (h - mu) / jnp.sqrt(var + 1e-5) + be_r[...]

    return pl.pallas_call(
        body,
        grid=(m // bm,),
        in_specs=[
            pl.BlockSpec((2, bm, feat + 1), lambda i: (0, i, 0)),
            pl.BlockSpec((bm, feat), lambda i: (i, 0)),
            pl.BlockSpec(wo.shape, lambda i: (0, 0)),
            pl.BlockSpec((1, feat), lambda i: (0, 0)),
            pl.BlockSpec((1, feat), lambda i: (0, 0)),
        ],
        out_specs=pl.BlockSpec((bm, feat), lambda i: (i, 0)),
        out_shape=jax.ShapeDtypeStruct((m, feat), jnp.float32),
    )(sc_out, x, wo, g[None, :], be[None, :])


def _atm_embed(l0, xl1, w1, b1, w2, b2, wl1, bl1, bm=1024):
    m = l0.shape[0]

    def body(l0_r, xl1_r, w1_r, b1_r, w2_r, b2_r, wl1_r, bl1_r, o_r):
        h = _elu(jnp.dot(l0_r[...], w1_r[...],
                         preferred_element_type=jnp.float32) + b1_r[...])
        h = jnp.dot(h, w2_r[...], preferred_element_type=jnp.float32) + b2_r[...]
        xv = xl1_r[...]
        nrm = jnp.sqrt(jnp.sum(xv * xv, axis=-1, keepdims=True))
        o_r[...] = h + nrm * wl1_r[...] + bl1_r[...]

    shapes = [l0.shape, xl1.shape, w1.shape, (1, b1.shape[0]), w2.shape,
              (1, b2.shape[0]), wl1.shape, (1, bl1.shape[0])]
    return pl.pallas_call(
        body,
        grid=(m // bm,),
        in_specs=_row_specs(m, bm, shapes),
        out_specs=pl.BlockSpec((bm, w2.shape[1]), lambda i: (i, 0)),
        out_shape=jax.ShapeDtypeStruct((m, w2.shape[1]), jnp.float32),
    )(l0, xl1, w1, b1[None, :], w2, b2[None, :], wl1, bl1[None, :])


def _r2a_mm(r2a_pad, h_res, bm=1024):
    m = r2a_pad.shape[0]
    kdim, feat = h_res.shape

    def body(a_r, h_r, o_r):
        o_r[...] = jnp.dot(a_r[...], h_r[...],
                           preferred_element_type=jnp.float32)

    return pl.pallas_call(
        body,
        grid=(m // bm,),
        in_specs=[pl.BlockSpec((bm, kdim), lambda i: (i, 0)),
                  pl.BlockSpec((kdim, feat), lambda i: (0, 0))],
        out_specs=pl.BlockSpec((bm, feat), lambda i: (i, 0)),
        out_shape=jax.ShapeDtypeStruct((m, feat), jnp.float32),
    )(r2a_pad, h_res)


def _a2r_mm(r2a_pad, h_atm, bk=1024):
    ka, nr = r2a_pad.shape
    feat = h_atm.shape[1]

    def body(a_r, h_r, o_r, acc_r):
        i = pl.program_id(0)

        @pl.when(i == 0)
        def _():
            acc_r[...] = jnp.zeros_like(acc_r)

        blk = a_r[...]
        hv = h_r[...]
        ones = jnp.ones((blk.shape[0], 1), jnp.float32)
        rhs = jnp.concatenate([hv, ones], axis=1)
        acc_r[...] += jax.lax.dot_general(
            blk, rhs, (((0,), (0,)), ((), ())),
            preferred_element_type=jnp.float32)

        @pl.when(i == (ka // bk) - 1)
        def _():
            acc = acc_r[...]
            o_r[...] = acc[:, :feat] / (acc[:, feat:feat + 1] + 1.0)

    return pl.pallas_call(
        body,
        grid=(ka // bk,),
        in_specs=[pl.BlockSpec((bk, nr), lambda i: (i, 0)),
                  pl.BlockSpec((bk, feat), lambda i: (i, 0))],
        out_specs=pl.BlockSpec((nr, feat), lambda i: (0, 0)),
        out_shape=jax.ShapeDtypeStruct((nr, feat), jnp.float32),
        scratch_shapes=[pltpu.VMEM((nr, feat + 1), jnp.float32)],
    )(r2a_pad, h_atm)


def _head(sc_out, h_atm, wskip, wc1, bc1, wc2, bc2, ww1, bw1, ww2, bw2):
    m = h_atm.shape[0]

    def body(sc_r, ha_r, wsk_r, wc1_r, bc1_r, wc2_r, bc2_r,
             ww1_r, bw1_r, ww2_r, bw2_r, o_r):
        acc = sc_r[0] + sc_r[1]
        h64 = acc[:, :64] / (acc[:, 64:65] + 1e-9)
        h64 = h64 + jnp.dot(ha_r[...], wsk_r[...],
                            preferred_element_type=jnp.float32)
        hc = jnp.maximum(jnp.dot(h64, wc1_r[...],
                                 preferred_element_type=jnp.float32)
                         + bc1_r[...], 0.0)
        c = jnp.dot(hc, wc2_r[...], preferred_element_type=jnp.float32) + bc2_r[...]
        hw = jnp.maximum(jnp.dot(h64, ww1_r[...],
                                 preferred_element_type=jnp.float32)
                         + bw1_r[...], 0.0)
        wl = jnp.dot(hw, ww2_r[...], preferred_element_type=jnp.float32) + bw2_r[...]
        rows = lax.broadcasted_iota(jnp.int32, (m, 1), 0)
        wl = jnp.where(rows < N_ATM, wl, -1e30)
        mx = jnp.max(wl)
        ex = jnp.exp(wl - mx)
        den = jnp.sum(ex)
        o_r[...] = jnp.sum(ex * c, axis=0, keepdims=True) / den

    return pl.pallas_call(
        body,
        in_specs=[pl.BlockSpec(sc_out.shape, lambda: (0, 0, 0))]
        + [pl.BlockSpec(s, lambda: (0,) * len(s))
           for s in [h_atm.shape, wskip.shape, wc1.shape, (1, 64), wc2.shape,
                     (1, 2), ww1.shape, (1, 64), ww2.shape, (1, 1)]],
        out_specs=pl.BlockSpec((1, 2), lambda: (0, 0)),
        out_shape=jax.ShapeDtypeStruct((1, 2), jnp.float32),
    )(sc_out, h_atm, wskip, wc1, bc1[None, :], wc2, bc2[None, :],
      ww1, bw1[None, :], ww2, bw2[None, :])


def _pad_rows(x, n):
    return jnp.pad(x, ((0, n - x.shape[0]),) + ((0, 0),) * (x.ndim - 1))


def _pad_edges(ei, dummy, n_chunks, chunk):
    e_pad = NW * n_chunks * chunk
    src = jnp.pad(ei[0], (0, e_pad - ei.shape[1]), constant_values=dummy)
    dst = jnp.pad(ei[1], (0, e_pad - ei.shape[1]), constant_values=dummy)
    return src.reshape(NW * n_chunks, chunk), dst.reshape(NW * n_chunks, chunk)


def _n_chunks(e, chunk):
    n = -(-e // (NW * chunk))
    return -(-n // 8) * 8


_SC_ATM32 = _edge_attn_sc(NA_PAD, 32, _n_chunks(640000, 128), 128)
_SC_RES32 = _edge_attn_sc(NR_PAD, 32, _n_chunks(16000, 128), 128)
_SC_ATM64 = _edge_attn_sc(NA_PAD, 64, _n_chunks(640000, 64), 64)


def _mp_layer(x, src2d, dst2d, lp, zeros, sc_fn):
    xm, xq = _premix(x, lp['Wm'], lp['Wq'])
    sc_out = sc_fn(xm, xq, src2d, dst2d, zeros)
    return _post(sc_out, x, lp['Wo'], lp['g'], lp['be'])


def kernel(x_bnd, x_res, x_atm_l1, edge_index_bnd, edge_index_atm,
           edge_index_res, r2a, params):
    p = params
    xb = _pad_rows(x_bnd, NA_PAD)
    xr = _pad_rows(x_res, NR_PAD)
    xl1 = _pad_rows(x_atm_l1.reshape(N_ATM, 3), NA_PAD)
    r2ap = jnp.pad(r2a, ((0, NA_PAD - N_ATM), (0, NR_PAD - N_RES)))

    srcb, dstb = _pad_edges(edge_index_bnd, N_ATM, _n_chunks(640000, 128), 128)
    srca, dsta = _pad_edges(edge_index_atm, N_ATM, _n_chunks(640000, 128), 128)
    srcr, dstr = _pad_edges(edge_index_res, N_RES, _n_chunks(16000, 128), 128)
    srcf, dstf = _pad_edges(edge_index_atm, N_ATM, _n_chunks(640000, 64), 64)

    z33a = jnp.zeros((NA_PAD, 33), jnp.float32)
    z33r = jnp.zeros((NR_PAD, 33), jnp.float32)
    z65a = jnp.zeros((NA_PAD, 65), jnp.float32)

    h_bnd = _mlp2(xb, p['W1_bnd'], p['b1_bnd'], p['W2_bnd'], p['b2_bnd'])
    for lp in p['bnd_layers']:
        h_bnd = _mp_layer(h_bnd, srcb, dstb, lp, z33a, _SC_ATM32)

    h_res = _mlp2(xr, p['W1_res'], p['b1_res'], p['W2_res'], p['b2_res'])
    h_resA = _r2a_mm(r2ap, h_res)
    l0 = jnp.concatenate([h_bnd, h_resA], axis=1)
    h_atm = _atm_embed(l0, xl1, p['W1_atm'], p['b1_atm'], p['W2_atm'],
                       p['b2_atm'], p['Wl1'], p['bl1'])

    for i, (lpa, lpr) in enumerate(zip(p['atm_layers'], p['res_layers'])):
        h_atm = _mp_layer(h_atm, srca, dsta, lpa, z33a, _SC_ATM32)
        h_res = _mp_layer(h_res, srcr, dstr, lpr, z33r, _SC_RES32)
        if i % 2 == 1:
            hA = jnp.concatenate([h_atm, _r2a_mm(r2ap, h_res)], axis=1)
            hR = jnp.concatenate([h_res, _a2r_mm(r2ap, h_atm)], axis=1)
            h_atm = _lin_elu(hA, p['Wla'], p['bla'])
            h_res = _lin_elu(hR, p['Wlr'], p['blr'])

    xm64, xq64 = _premix(h_atm, p['fin']['Wm'], p['fin']['Wq'])
    sc65 = _SC_ATM64(xm64, xq64, srcf, dstf, z65a)
    out = _head(sc65, h_atm, p['Wskip'], p['Wc1'], p['bc1'], p['Wc2'],
                p['bc2'], p['Ww1'], p['bw1'], p['Ww2'], p['bw2'])
    return out.reshape(2)
